# baseline (speedup 1.0000x reference)
import functools

import jax
import jax.numpy as jnp
from jax import lax
from jax.experimental import pallas as pl
from jax.experimental.pallas import tpu as pltpu
from jax.experimental.pallas import tpu_sc as plsc

_NBUF = 2
_CHUNK = 1600


@functools.cache
def _build(n, V, D, dtype):
    info = plsc.get_sparse_core_info()
    NC, NS = info.num_cores, info.num_subcores
    NW = NC * NS
    assert n % NW == 0
    b_per_w = n // NW
    C = _CHUNK
    while b_per_w % C:
        C //= 2
    n_chunks = b_per_w // C

    mesh = plsc.VectorSubcoreMesh(core_axis_name="c", subcore_axis_name="s")

    @functools.partial(
        pl.kernel,
        out_type=jax.ShapeDtypeStruct((n, D), dtype),
        mesh=mesh,
        scratch_types=[
            pltpu.VMEM((b_per_w,), jnp.int32),
            pltpu.VMEM((_NBUF, C, D), dtype),
            pltpu.SemaphoreType.DMA,
            pltpu.SemaphoreType.DMA,
        ],
        compiler_params=pltpu.CompilerParams(use_tc_tiling_on_sc=False),
    )
    def k(idx_hbm, table_hbm, out_hbm, idx_v, rows_v, gsem, ssem):
        wid = lax.axis_index("s") * NC + lax.axis_index("c")
        base = wid * b_per_w
        pltpu.sync_copy(idx_hbm.at[pl.ds(base, b_per_w)], idx_v)

        def start_gather(g):
            buf = lax.rem(g, _NBUF)
            pltpu.async_copy(
                table_hbm.at[idx_v.at[pl.ds(g * C, C)]], rows_v.at[buf], gsem
            )

        def wait_gather():
            pltpu.make_async_copy(out_hbm.at[pl.ds(0, C)], rows_v.at[0], gsem).wait()

        def start_store(g):
            buf = lax.rem(g, _NBUF)
            pltpu.async_copy(rows_v.at[buf], out_hbm.at[pl.ds(base + g * C, C)], ssem)

        def wait_store():
            pltpu.make_async_copy(out_hbm.at[pl.ds(0, C)], rows_v.at[0], ssem).wait()

        for b in range(_NBUF):
            start_gather(b)

        def body(g, carry):
            wait_gather()
            start_store(g)

            @pl.when(g + _NBUF < n_chunks)
            def _():
                wait_store()
                start_gather(g + _NBUF)

            return carry

        lax.fori_loop(0, n_chunks, body, 0)
        for _ in range(_NBUF):
            wait_store()

    return k


def kernel(x, table):
    B, L = x.shape
    V, D = table.shape
    n = B * L
    flat = x.reshape(n)
    out = _build(n, V, D, table.dtype)(flat, table)
    return out.reshape(B, L, D)


# speedup vs baseline: 1.6436x; 1.6436x over previous
import functools

import jax
import jax.numpy as jnp
from jax import lax
from jax.experimental import pallas as pl
from jax.experimental.pallas import tpu as pltpu
from jax.experimental.pallas import tpu_sc as plsc

_NBUF = 2
_CB = 32


@functools.cache
def _build(B, L, V, D, dtype):
    info = plsc.get_sparse_core_info()
    NC, NS = info.num_cores, info.num_subcores
    NW = NC * NS
    assert B % NW == 0
    rows_per_w = B // NW
    CB = _CB
    while rows_per_w % CB:
        CB //= 2
    n_chunks = rows_per_w // CB

    mesh = plsc.VectorSubcoreMesh(core_axis_name="c", subcore_axis_name="s")

    @functools.partial(
        pl.kernel,
        out_type=jax.ShapeDtypeStruct((B, L, D), dtype),
        mesh=mesh,
        scratch_types=[
            pltpu.VMEM((rows_per_w, L), jnp.int32),
            pltpu.VMEM((_NBUF, CB, L, D), dtype),
            pltpu.SemaphoreType.DMA,
            pltpu.SemaphoreType.DMA,
        ],
        compiler_params=pltpu.CompilerParams(use_tc_tiling_on_sc=False),
    )
    def k(x_hbm, table_hbm, out_hbm, idx_v, rows_v, gsem, ssem):
        wid = lax.axis_index("s") * NC + lax.axis_index("c")
        base_b = wid * rows_per_w
        pltpu.sync_copy(x_hbm.at[pl.ds(base_b, rows_per_w)], idx_v)

        def start_gathers(g):
            buf = lax.rem(g, _NBUF)

            def one(j, carry):
                pltpu.async_copy(
                    table_hbm.at[idx_v.at[g * CB + j]],
                    rows_v.at[buf].at[j],
                    gsem,
                )
                return carry

            lax.fori_loop(0, CB, one, 0)

        def wait_gathers():
            pltpu.make_async_copy(out_hbm.at[pl.ds(0, CB)], rows_v.at[0], gsem).wait()

        def start_store(g):
            buf = lax.rem(g, _NBUF)
            pltpu.async_copy(
                rows_v.at[buf], out_hbm.at[pl.ds(base_b + g * CB, CB)], ssem
            )

        def wait_store():
            pltpu.make_async_copy(out_hbm.at[pl.ds(0, CB)], rows_v.at[0], ssem).wait()

        for b in range(_NBUF):
            start_gathers(b)

        def body(g, carry):
            wait_gathers()
            start_store(g)

            @pl.when(g + _NBUF < n_chunks)
            def _():
                wait_store()
                start_gathers(g + _NBUF)

            return carry

        lax.fori_loop(0, n_chunks, body, 0)
        for _ in range(_NBUF):
            wait_store()

    return k


def kernel(x, table):
    B, L = x.shape
    V, D = table.shape
    return _build(B, L, V, D, table.dtype)(x, table)
